# Initial kernel scaffold; baseline (speedup 1.0000x reference)
import jax
import jax.numpy as jnp
from jax.experimental import pallas as pl


def kernel(x, edge_index, edge_attr, W, b):
    raise NotImplementedError("write your pallas kernel here")


# speedup vs baseline: 9.2929x; 9.2929x over previous
import functools

import jax
import jax.numpy as jnp
import numpy as np
from jax import lax
from jax.experimental import pallas as pl
from jax.experimental.pallas import tpu as pltpu
from jax.experimental.pallas import tpu_sc as plsc

_KEEP_EDGE = 0.5
_DIM_IN = 128
_DIM_OUT = 128
_EDGE_DIM = 16
_N_NODES = 10000
_N_EDGES = 320000


def _rotl32(x, r):
    return ((x << np.uint32(r)) | (x >> np.uint32(32 - r))).astype(np.uint32)


def _threefry2x32(k, x0, x1):
    k0, k1 = np.uint32(k[0]), np.uint32(k[1])
    ks = [k0, k1, np.uint32(k0 ^ k1 ^ np.uint32(0x1BD11BDA))]
    rot = [[13, 15, 26, 6], [17, 29, 16, 24]]
    x0 = (x0 + ks[0]).astype(np.uint32)
    x1 = (x1 + ks[1]).astype(np.uint32)
    for i in range(5):
        for r in rot[i % 2]:
            x0 = (x0 + x1).astype(np.uint32)
            x1 = _rotl32(x1, r)
            x1 = (x1 ^ x0).astype(np.uint32)
        x0 = (x0 + ks[(i + 1) % 3]).astype(np.uint32)
        x1 = (x1 + ks[(i + 2) % 3] + np.uint32(i + 1)).astype(np.uint32)
    return x0, x1


def _edge_mask(n, fold_data, thresh):
    fk0, fk1 = _threefry2x32(
        (np.uint32(0), np.uint32(0)), np.uint32([0]), np.uint32([fold_data])
    )
    o0, o1 = _threefry2x32(
        (fk0[0], fk1[0]), np.zeros(n, np.uint32), np.arange(n, dtype=np.uint32)
    )
    bits = (o0 ^ o1).astype(np.uint32)
    f = ((bits >> np.uint32(9)) | np.uint32(0x3F800000)).view(np.float32)
    return (f - np.float32(1.0)) < thresh


_MASK_NP = _edge_mask(_N_EDGES, 12345, _KEEP_EDGE)
_N_KEPT = int(_MASK_NP.sum())
_KEPT_NP = np.nonzero(_MASK_NP)[0].astype(np.int32)

_NW = 32
_K = 128
_E_PAD = ((_N_KEPT + _NW * _K - 1) // (_NW * _K)) * (_NW * _K)
_NBLK = _E_PAD // _K
_CHUNKS = _NBLK // _NW
_N_ACC = 10112
_ROWS_PER_SUB = _N_ACC // 16

_KI_PAD_NP = np.concatenate(
    [_KEPT_NP, np.zeros(_E_PAD - _N_KEPT, np.int32)]
)


def _h_matmul(x, w1, b2d):
    def body(x_ref, w_ref, b_ref, o_ref):
        o_ref[...] = (
            jnp.dot(x_ref[...], w_ref[...], preferred_element_type=jnp.float32)
            + b_ref[...]
        )

    return pl.pallas_call(
        body,
        grid=(10,),
        in_specs=[
            pl.BlockSpec((1000, 128), lambda i: (i, 0)),
            pl.BlockSpec((128, 128), lambda i: (0, 0)),
            pl.BlockSpec((1, 128), lambda i: (0, 0)),
        ],
        out_specs=pl.BlockSpec((1000, 128), lambda i: (i, 0)),
        out_shape=jax.ShapeDtypeStruct((_N_NODES, 128), jnp.float32),
    )(x, w1, b2d)


def _combine(hacc, eaacc, w2):
    def body(h_ref, ea_ref, w_ref, o_ref):
        ea = ea_ref[0] + ea_ref[1]
        o_ref[...] = (
            h_ref[0]
            + h_ref[1]
            + jnp.dot(ea, w_ref[...], preferred_element_type=jnp.float32)
        )

    return pl.pallas_call(
        body,
        grid=(10,),
        in_specs=[
            pl.BlockSpec((2, 1000, 128), lambda i: (0, i, 0)),
            pl.BlockSpec((2, 1000, 16), lambda i: (0, i, 0)),
            pl.BlockSpec((16, 128), lambda i: (0, 0)),
        ],
        out_specs=pl.BlockSpec((1000, 128), lambda i: (i, 0)),
        out_shape=jax.ShapeDtypeStruct((_N_NODES, 128), jnp.float32),
    )(hacc, eaacc, w2)


def _sc_aggregate(h, edge_attr, idx, zh, zea):
    mesh = plsc.VectorSubcoreMesh(core_axis_name="c", subcore_axis_name="s")

    @functools.partial(
        pl.kernel,
        out_type=(
            jax.ShapeDtypeStruct((2, _N_ACC, 128), jnp.float32),
            jax.ShapeDtypeStruct((2, _N_ACC, 16), jnp.float32),
        ),
        mesh=mesh,
        scratch_types=[
            pltpu.VMEM((3, _K), jnp.int32),
            pltpu.VMEM((_K, 128), jnp.float32),
            pltpu.VMEM((_K, 16), jnp.float32),
            pltpu.VMEM_SHARED((_N_ACC, 128), jnp.float32),
            pltpu.VMEM_SHARED((_N_ACC, 16), jnp.float32),
            pltpu.SemaphoreType.DMA,
            pltpu.SemaphoreType.DMA,
        ],
        compiler_params=pltpu.CompilerParams(use_tc_tiling_on_sc=False),
    )
    def k(h_hbm, ea_hbm, idx_hbm, zh_hbm, zea_hbm, outh_hbm, outea_hbm,
          idxv, hrows, earows, hacc, eaacc, sem1, sem2):
        c = lax.axis_index("c")
        s = lax.axis_index("s")
        w = c * 16 + s
        rz = s * _ROWS_PER_SUB
        pltpu.sync_copy(
            zh_hbm.at[pl.ds(rz, _ROWS_PER_SUB)], hacc.at[pl.ds(rz, _ROWS_PER_SUB)]
        )
        pltpu.sync_copy(
            zea_hbm.at[pl.ds(rz, _ROWS_PER_SUB)], eaacc.at[pl.ds(rz, _ROWS_PER_SUB)]
        )
        plsc.subcore_barrier()

        @pl.loop(0, _CHUNKS)
        def _(j):
            blk = w * _CHUNKS + j
            pltpu.sync_copy(idx_hbm.at[blk], idxv)
            cp1 = pltpu.async_copy(h_hbm.at[idxv.at[0]], hrows, sem1)
            cp2 = pltpu.async_copy(ea_hbm.at[idxv.at[2]], earows, sem2)
            cp1.wait()
            cp2.wait()
            pltpu.sync_copy(hrows, hacc.at[idxv.at[1]], add=True)
            pltpu.sync_copy(earows, eaacc.at[idxv.at[1]], add=True)

        plsc.subcore_barrier()
        pltpu.sync_copy(
            hacc.at[pl.ds(rz, _ROWS_PER_SUB)],
            outh_hbm.at[c, pl.ds(rz, _ROWS_PER_SUB)],
        )
        pltpu.sync_copy(
            eaacc.at[pl.ds(rz, _ROWS_PER_SUB)],
            outea_hbm.at[c, pl.ds(rz, _ROWS_PER_SUB)],
        )

    return k(h, edge_attr, idx, zh, zea)


def kernel(x, edge_index, edge_attr, W, b):
    w1 = W[:_DIM_IN]
    w2 = W[_DIM_IN:]
    b2d = b.reshape(1, _DIM_OUT)

    h = _h_matmul(x, w1, b2d)

    kept = jnp.asarray(_KEPT_NP)
    ei = jnp.take(edge_index, kept, axis=1)

    pad = _E_PAD - _N_KEPT
    src_pad = jnp.concatenate([ei[0], jnp.zeros((pad,), jnp.int32)])
    dst_pad = jnp.concatenate([ei[1], jnp.full((pad,), _N_NODES, jnp.int32)])
    ki_pad = jnp.asarray(_KI_PAD_NP)
    idx = jnp.stack([src_pad, dst_pad, ki_pad], axis=0)
    idx = idx.reshape(3, _NBLK, _K).transpose(1, 0, 2)

    zh = jnp.zeros((_N_ACC, 128), jnp.float32)
    zea = jnp.zeros((_N_ACC, 16), jnp.float32)

    hacc, eaacc = _sc_aggregate(h, edge_attr, idx, zh, zea)
    out = _combine(hacc, eaacc, w2)
    return (out, ei, edge_attr)


# speedup vs baseline: 9.6960x; 1.0434x over previous
import functools

import jax
import jax.numpy as jnp
import numpy as np
from jax import lax
from jax.experimental import pallas as pl
from jax.experimental.pallas import tpu as pltpu
from jax.experimental.pallas import tpu_sc as plsc

_KEEP_EDGE = 0.5
_DIM_IN = 128
_DIM_OUT = 128
_EDGE_DIM = 16
_N_NODES = 10000
_N_EDGES = 320000


def _rotl32(x, r):
    return ((x << np.uint32(r)) | (x >> np.uint32(32 - r))).astype(np.uint32)


def _threefry2x32(k, x0, x1):
    k0, k1 = np.uint32(k[0]), np.uint32(k[1])
    ks = [k0, k1, np.uint32(k0 ^ k1 ^ np.uint32(0x1BD11BDA))]
    rot = [[13, 15, 26, 6], [17, 29, 16, 24]]
    x0 = (x0 + ks[0]).astype(np.uint32)
    x1 = (x1 + ks[1]).astype(np.uint32)
    for i in range(5):
        for r in rot[i % 2]:
            x0 = (x0 + x1).astype(np.uint32)
            x1 = _rotl32(x1, r)
            x1 = (x1 ^ x0).astype(np.uint32)
        x0 = (x0 + ks[(i + 1) % 3]).astype(np.uint32)
        x1 = (x1 + ks[(i + 2) % 3] + np.uint32(i + 1)).astype(np.uint32)
    return x0, x1


def _edge_mask(n, fold_data, thresh):
    fk0, fk1 = _threefry2x32(
        (np.uint32(0), np.uint32(0)), np.uint32([0]), np.uint32([fold_data])
    )
    o0, o1 = _threefry2x32(
        (fk0[0], fk1[0]), np.zeros(n, np.uint32), np.arange(n, dtype=np.uint32)
    )
    bits = (o0 ^ o1).astype(np.uint32)
    f = ((bits >> np.uint32(9)) | np.uint32(0x3F800000)).view(np.float32)
    return (f - np.float32(1.0)) < thresh


_MASK_NP = _edge_mask(_N_EDGES, 12345, _KEEP_EDGE)
_N_KEPT = int(_MASK_NP.sum())
_KEPT_NP = np.nonzero(_MASK_NP)[0].astype(np.int32)

_NW = 32
_K = 64
_NBUF = 4
_E_PAD = ((_N_KEPT + _NW * _K * _NBUF - 1) // (_NW * _K * _NBUF)) * (_NW * _K * _NBUF)
_NBLK = _E_PAD // _K
_CHUNKS = _NBLK // _NW
_N_ACC = 10112
_ROWS_PER_SUB = _N_ACC // 16

_KI_PAD_NP = np.concatenate(
    [_KEPT_NP, np.zeros(_E_PAD - _N_KEPT, np.int32)]
)


def _h_matmul(x, w1, b2d):
    def body(x_ref, w_ref, b_ref, o_ref):
        o_ref[...] = (
            jnp.dot(x_ref[...], w_ref[...], preferred_element_type=jnp.float32)
            + b_ref[...]
        )

    return pl.pallas_call(
        body,
        grid=(10,),
        in_specs=[
            pl.BlockSpec((1000, 128), lambda i: (i, 0)),
            pl.BlockSpec((128, 128), lambda i: (0, 0)),
            pl.BlockSpec((1, 128), lambda i: (0, 0)),
        ],
        out_specs=pl.BlockSpec((1000, 128), lambda i: (i, 0)),
        out_shape=jax.ShapeDtypeStruct((_N_NODES, 128), jnp.float32),
    )(x, w1, b2d)


def _combine(hacc, eaacc, w2):
    def body(h_ref, ea_ref, w_ref, o_ref):
        ea = ea_ref[0] + ea_ref[1]
        o_ref[...] = (
            h_ref[0]
            + h_ref[1]
            + jnp.dot(ea, w_ref[...], preferred_element_type=jnp.float32)
        )

    return pl.pallas_call(
        body,
        grid=(10,),
        in_specs=[
            pl.BlockSpec((2, 1000, 128), lambda i: (0, i, 0)),
            pl.BlockSpec((2, 1000, 16), lambda i: (0, i, 0)),
            pl.BlockSpec((16, 128), lambda i: (0, 0)),
        ],
        out_specs=pl.BlockSpec((1000, 128), lambda i: (i, 0)),
        out_shape=jax.ShapeDtypeStruct((_N_NODES, 128), jnp.float32),
    )(hacc, eaacc, w2)


_NB = _NBUF


def _sc_aggregate(h, edge_attr, idx, zh, zea, dum):
    mesh = plsc.VectorSubcoreMesh(core_axis_name="c", subcore_axis_name="s")
    assert _CHUNKS % _NB == 0

    @functools.partial(
        pl.kernel,
        out_type=(
            jax.ShapeDtypeStruct((2, _N_ACC, 128), jnp.float32),
            jax.ShapeDtypeStruct((2, _N_ACC, 16), jnp.float32),
        ),
        mesh=mesh,
        scratch_types=(
            [pltpu.VMEM((3, _K), jnp.int32) for _ in range(_NB)]
            + [pltpu.VMEM((_K, 128), jnp.float32) for _ in range(_NB)]
            + [pltpu.VMEM((_K, 16), jnp.float32) for _ in range(_NB)]
            + [
                pltpu.VMEM((_K,), jnp.int32),
                pltpu.VMEM_SHARED((_N_ACC, 128), jnp.float32),
                pltpu.VMEM_SHARED((_N_ACC, 16), jnp.float32),
            ]
            + [pltpu.SemaphoreType.DMA for _ in range(3 * _NB)]
        ),
        compiler_params=pltpu.CompilerParams(use_tc_tiling_on_sc=False),
    )
    def k(h_hbm, ea_hbm, idx_hbm, zh_hbm, zea_hbm, dum_hbm, outh_hbm, outea_hbm,
          *scratch):
        idxv = scratch[0:_NB]
        hrows = scratch[_NB:2 * _NB]
        earows = scratch[2 * _NB:3 * _NB]
        dummyv = scratch[3 * _NB]
        hacc = scratch[3 * _NB + 1]
        eaacc = scratch[3 * _NB + 2]
        isem = scratch[3 * _NB + 3:3 * _NB + 3 + _NB]
        gsem = scratch[3 * _NB + 3 + _NB:3 * _NB + 3 + 2 * _NB]
        ssem = scratch[3 * _NB + 3 + 2 * _NB:]

        c = lax.axis_index("c")
        s = lax.axis_index("s")
        w = c * 16 + s
        rz = s * _ROWS_PER_SUB

        def issue_idx(b, blk):
            pltpu.async_copy(idx_hbm.at[blk], idxv[b], isem[b])

        def wait_idx(b):
            pltpu.make_async_copy(idx_hbm.at[0], idxv[b], isem[b]).wait()

        def issue_gathers(b):
            pltpu.async_copy(h_hbm.at[idxv[b].at[0]], hrows[b], gsem[b])
            pltpu.async_copy(ea_hbm.at[idxv[b].at[2]], earows[b], gsem[b])

        def wait_gathers(b):
            pltpu.make_async_copy(h_hbm.at[idxv[b].at[0]], hrows[b], gsem[b]).wait()
            pltpu.make_async_copy(ea_hbm.at[idxv[b].at[2]], earows[b], gsem[b]).wait()

        def issue_scatters(b):
            pltpu.async_copy(hrows[b], hacc.at[idxv[b].at[1]], ssem[b], add=True)
            pltpu.async_copy(earows[b], eaacc.at[idxv[b].at[1]], ssem[b], add=True)

        def wait_scatters(b):
            pltpu.make_async_copy(hrows[b], hacc.at[idxv[b].at[1]], ssem[b]).wait()
            pltpu.make_async_copy(earows[b], eaacc.at[idxv[b].at[1]], ssem[b]).wait()

        def dummy_scatters(b):
            pltpu.async_copy(hrows[b], hacc.at[dummyv], ssem[b], add=True)
            pltpu.async_copy(earows[b], eaacc.at[dummyv], ssem[b], add=True)

        pltpu.sync_copy(
            zh_hbm.at[pl.ds(rz, _ROWS_PER_SUB)], hacc.at[pl.ds(rz, _ROWS_PER_SUB)]
        )
        pltpu.sync_copy(
            zea_hbm.at[pl.ds(rz, _ROWS_PER_SUB)], eaacc.at[pl.ds(rz, _ROWS_PER_SUB)]
        )
        pltpu.sync_copy(dum_hbm, dummyv)
        plsc.subcore_barrier()

        base = w * _CHUNKS
        for b in range(_NB):
            dummy_scatters(b)
        issue_idx(0, base)
        issue_idx(1, base + 1)
        wait_idx(0)
        issue_gathers(0)

        @pl.loop(0, _CHUNKS, step=_NB)
        def _(j0):
            for b in range(_NB):
                j = j0 + b
                b1 = (b + 1) % _NB
                b2 = (b + 2) % _NB
                wait_gathers(b)
                issue_scatters(b)
                wait_scatters(b2)
                issue_idx(b2, base + j + 2)
                wait_idx(b1)
                issue_gathers(b1)

        for b in range(_NB):
            wait_scatters(b)
        wait_gathers(0)
        wait_idx(1)
        plsc.subcore_barrier()
        pltpu.sync_copy(
            hacc.at[pl.ds(rz, _ROWS_PER_SUB)],
            outh_hbm.at[c, pl.ds(rz, _ROWS_PER_SUB)],
        )
        pltpu.sync_copy(
            eaacc.at[pl.ds(rz, _ROWS_PER_SUB)],
            outea_hbm.at[c, pl.ds(rz, _ROWS_PER_SUB)],
        )

    return k(h, edge_attr, idx, zh, zea, dum)


def kernel(x, edge_index, edge_attr, W, b):
    w1 = W[:_DIM_IN]
    w2 = W[_DIM_IN:]
    b2d = b.reshape(1, _DIM_OUT)

    h = _h_matmul(x, w1, b2d)

    kept = jnp.asarray(_KEPT_NP)
    ei = jnp.take(edge_index, kept, axis=1)

    pad = _E_PAD - _N_KEPT
    src_pad = jnp.concatenate([ei[0], jnp.zeros((pad,), jnp.int32)])
    dst_pad = jnp.concatenate([ei[1], jnp.full((pad,), _N_NODES, jnp.int32)])
    ki_pad = jnp.asarray(_KI_PAD_NP)
    idx = jnp.stack([src_pad, dst_pad, ki_pad], axis=0)
    idx = idx.reshape(3, _NBLK, _K).transpose(1, 0, 2)
    extra = np.zeros((2, 3, _K), np.int32)
    extra[:, 1, :] = _N_NODES
    idx = jnp.concatenate([idx, jnp.asarray(extra)], axis=0)

    zh = jnp.zeros((_N_ACC, 128), jnp.float32)
    zea = jnp.zeros((_N_ACC, 16), jnp.float32)
    dum = jnp.full((_K,), _N_NODES, jnp.int32)

    hacc, eaacc = _sc_aggregate(h, edge_attr, idx, zh, zea, dum)
    out = _combine(hacc, eaacc, w2)
    return (out, ei, edge_attr)


# speedup vs baseline: 9.9993x; 1.0313x over previous
import functools

import jax
import jax.numpy as jnp
import numpy as np
from jax import lax
from jax.experimental import pallas as pl
from jax.experimental.pallas import tpu as pltpu
from jax.experimental.pallas import tpu_sc as plsc

_KEEP_EDGE = 0.5
_DIM_IN = 128
_DIM_OUT = 128
_EDGE_DIM = 16
_N_NODES = 10000
_N_EDGES = 320000


def _rotl32(x, r):
    return ((x << np.uint32(r)) | (x >> np.uint32(32 - r))).astype(np.uint32)


def _threefry2x32(k, x0, x1):
    k0, k1 = np.uint32(k[0]), np.uint32(k[1])
    ks = [k0, k1, np.uint32(k0 ^ k1 ^ np.uint32(0x1BD11BDA))]
    rot = [[13, 15, 26, 6], [17, 29, 16, 24]]
    x0 = (x0 + ks[0]).astype(np.uint32)
    x1 = (x1 + ks[1]).astype(np.uint32)
    for i in range(5):
        for r in rot[i % 2]:
            x0 = (x0 + x1).astype(np.uint32)
            x1 = _rotl32(x1, r)
            x1 = (x1 ^ x0).astype(np.uint32)
        x0 = (x0 + ks[(i + 1) % 3]).astype(np.uint32)
        x1 = (x1 + ks[(i + 2) % 3] + np.uint32(i + 1)).astype(np.uint32)
    return x0, x1


def _edge_mask(n, fold_data, thresh):
    fk0, fk1 = _threefry2x32(
        (np.uint32(0), np.uint32(0)), np.uint32([0]), np.uint32([fold_data])
    )
    o0, o1 = _threefry2x32(
        (fk0[0], fk1[0]), np.zeros(n, np.uint32), np.arange(n, dtype=np.uint32)
    )
    bits = (o0 ^ o1).astype(np.uint32)
    f = ((bits >> np.uint32(9)) | np.uint32(0x3F800000)).view(np.float32)
    return (f - np.float32(1.0)) < thresh


_MASK_NP = _edge_mask(_N_EDGES, 12345, _KEEP_EDGE)
_N_KEPT = int(_MASK_NP.sum())
_KEPT_NP = np.nonzero(_MASK_NP)[0].astype(np.int32)

_NW = 32
_K = 64
_NBUF = 4
_E_PAD = ((_N_KEPT + _NW * _K * _NBUF - 1) // (_NW * _K * _NBUF)) * (_NW * _K * _NBUF)
_NBLK = _E_PAD // _K
_CHUNKS = _NBLK // _NW
_N_ACC = 10112
_ROWS_PER_SUB = _N_ACC // 16

_KI_PAD_NP = np.concatenate(
    [_KEPT_NP, np.zeros(_E_PAD - _N_KEPT, np.int32)]
)


def _h_matmul(x, w1, b2d):
    def body(x_ref, w_ref, b_ref, o_ref):
        o_ref[...] = (
            jnp.dot(x_ref[...], w_ref[...], preferred_element_type=jnp.float32)
            + b_ref[...]
        )

    return pl.pallas_call(
        body,
        grid=(10,),
        in_specs=[
            pl.BlockSpec((1000, 128), lambda i: (i, 0)),
            pl.BlockSpec((128, 128), lambda i: (0, 0)),
            pl.BlockSpec((1, 128), lambda i: (0, 0)),
        ],
        out_specs=pl.BlockSpec((1000, 128), lambda i: (i, 0)),
        out_shape=jax.ShapeDtypeStruct((_N_NODES, 128), jnp.float32),
    )(x, w1, b2d)


def _combine(hacc, eaacc, w2):
    def body(h_ref, ea_ref, w_ref, o_ref):
        ea = ea_ref[0] + ea_ref[1]
        o_ref[...] = (
            h_ref[0]
            + h_ref[1]
            + jnp.dot(ea, w_ref[...], preferred_element_type=jnp.float32)
        )

    return pl.pallas_call(
        body,
        grid=(10,),
        in_specs=[
            pl.BlockSpec((2, 1000, 128), lambda i: (0, i, 0)),
            pl.BlockSpec((2, 1000, 16), lambda i: (0, i, 0)),
            pl.BlockSpec((16, 128), lambda i: (0, 0)),
        ],
        out_specs=pl.BlockSpec((1000, 128), lambda i: (i, 0)),
        out_shape=jax.ShapeDtypeStruct((_N_NODES, 128), jnp.float32),
    )(hacc, eaacc, w2)


_NB = _NBUF


def _sc_aggregate(h, edge_attr, idx, zh, zea, dum):
    mesh = plsc.VectorSubcoreMesh(core_axis_name="c", subcore_axis_name="s")
    assert _CHUNKS % _NB == 0

    @functools.partial(
        pl.kernel,
        out_type=(
            jax.ShapeDtypeStruct((2, _N_ACC, 128), jnp.float32),
            jax.ShapeDtypeStruct((2, _N_ACC, 16), jnp.float32),
        ),
        mesh=mesh,
        scratch_types=(
            [pltpu.VMEM((3, _K), jnp.int32) for _ in range(_NB)]
            + [pltpu.VMEM((_K, 128), jnp.float32) for _ in range(_NB)]
            + [pltpu.VMEM((_K, 16), jnp.float32) for _ in range(_NB)]
            + [
                pltpu.VMEM((_K,), jnp.int32),
                pltpu.VMEM_SHARED((_N_ACC, 128), jnp.float32),
                pltpu.VMEM_SHARED((_N_ACC, 16), jnp.float32),
            ]
            + [pltpu.SemaphoreType.DMA for _ in range(3 * _NB)]
        ),
        compiler_params=pltpu.CompilerParams(use_tc_tiling_on_sc=False),
    )
    def k(h_hbm, ea_hbm, idx_hbm, zh_hbm, zea_hbm, dum_hbm, outh_hbm, outea_hbm,
          *scratch):
        idxv = scratch[0:_NB]
        hrows = scratch[_NB:2 * _NB]
        earows = scratch[2 * _NB:3 * _NB]
        dummyv = scratch[3 * _NB]
        hacc = scratch[3 * _NB + 1]
        eaacc = scratch[3 * _NB + 2]
        isem = scratch[3 * _NB + 3:3 * _NB + 3 + _NB]
        gsem = scratch[3 * _NB + 3 + _NB:3 * _NB + 3 + 2 * _NB]
        ssem = scratch[3 * _NB + 3 + 2 * _NB:]

        c = lax.axis_index("c")
        s = lax.axis_index("s")
        w = c * 16 + s
        rz = s * _ROWS_PER_SUB

        def issue_idx(b, blk):
            pltpu.async_copy(idx_hbm.at[blk], idxv[b], isem[b])

        def wait_idx(b):
            pltpu.make_async_copy(idx_hbm.at[0], idxv[b], isem[b]).wait()

        def issue_gathers(b):
            pltpu.async_copy(h_hbm.at[idxv[b].at[0]], hrows[b], gsem[b])
            pltpu.async_copy(ea_hbm.at[idxv[b].at[2]], earows[b], gsem[b])

        def wait_gathers(b):
            pltpu.make_async_copy(h_hbm.at[idxv[b].at[0]], hrows[b], gsem[b]).wait()
            pltpu.make_async_copy(ea_hbm.at[idxv[b].at[2]], earows[b], gsem[b]).wait()

        def issue_scatters(b):
            pltpu.async_copy(hrows[b], hacc.at[idxv[b].at[1]], ssem[b], add=True)
            pltpu.async_copy(earows[b], eaacc.at[idxv[b].at[1]], ssem[b], add=True)

        def wait_scatters(b):
            pltpu.make_async_copy(hrows[b], hacc.at[idxv[b].at[1]], ssem[b]).wait()
            pltpu.make_async_copy(earows[b], eaacc.at[idxv[b].at[1]], ssem[b]).wait()

        def dummy_scatters(b):
            pltpu.async_copy(hrows[b], hacc.at[dummyv], ssem[b], add=True)
            pltpu.async_copy(earows[b], eaacc.at[dummyv], ssem[b], add=True)

        pltpu.sync_copy(
            zh_hbm.at[pl.ds(rz, _ROWS_PER_SUB)], hacc.at[pl.ds(rz, _ROWS_PER_SUB)]
        )
        pltpu.sync_copy(
            zea_hbm.at[pl.ds(rz, _ROWS_PER_SUB)], eaacc.at[pl.ds(rz, _ROWS_PER_SUB)]
        )
        pltpu.sync_copy(dum_hbm, dummyv)
        plsc.subcore_barrier()

        base = w * _CHUNKS
        for b in range(_NB):
            dummy_scatters(b)
        issue_idx(0, base)
        issue_idx(1, base + 1)
        wait_idx(0)
        issue_gathers(0)

        @pl.loop(0, _CHUNKS, step=_NB)
        def _(j0):
            for b in range(_NB):
                j = j0 + b
                b1 = (b + 1) % _NB
                b2 = (b + 2) % _NB
                wait_gathers(b)
                issue_scatters(b)
                wait_scatters(b2)
                issue_idx(b2, base + j + 2)
                wait_idx(b1)
                issue_gathers(b1)

        for b in range(_NB):
            wait_scatters(b)
        wait_gathers(0)
        wait_idx(1)
        plsc.subcore_barrier()
        pltpu.sync_copy(
            hacc.at[pl.ds(rz, _ROWS_PER_SUB)],
            outh_hbm.at[c, pl.ds(rz, _ROWS_PER_SUB)],
        )
        pltpu.sync_copy(
            eaacc.at[pl.ds(rz, _ROWS_PER_SUB)],
            outea_hbm.at[c, pl.ds(rz, _ROWS_PER_SUB)],
        )

    return k(h, edge_attr, idx, zh, zea, dum)


def kernel(x, edge_index, edge_attr, W, b):
    w1 = W[:_DIM_IN]
    w2 = W[_DIM_IN:]
    b2d = b.reshape(1, _DIM_OUT)

    h = _h_matmul(x, w1, b2d)

    kept = jnp.asarray(_KEPT_NP)
    ei = jnp.take(edge_index, kept, axis=1)

    pad = _E_PAD - _N_KEPT
    dummy_dst = _N_NODES + (np.arange(pad, dtype=np.int32) % (_N_ACC - _N_NODES))
    src_pad = jnp.concatenate([ei[0], jnp.zeros((pad,), jnp.int32)])
    dst_pad = jnp.concatenate([ei[1], jnp.asarray(dummy_dst)])
    ki_pad = jnp.asarray(_KI_PAD_NP)
    idx = jnp.stack([src_pad, dst_pad, ki_pad], axis=0)
    idx = idx.reshape(3, _NBLK, _K).transpose(1, 0, 2)
    extra = np.zeros((2, 3, _K), np.int32)
    extra[:, 1, :] = _N_NODES + (np.arange(2 * _K, dtype=np.int32).reshape(2, _K)
                                 % (_N_ACC - _N_NODES))
    idx = jnp.concatenate([idx, jnp.asarray(extra)], axis=0)

    zh = jnp.zeros((_N_ACC, 128), jnp.float32)
    zea = jnp.zeros((_N_ACC, 16), jnp.float32)
    dum = jnp.asarray(
        _N_NODES + (np.arange(_K, dtype=np.int32) % (_N_ACC - _N_NODES))
    )

    hacc, eaacc = _sc_aggregate(h, edge_attr, idx, zh, zea, dum)
    out = _combine(hacc, eaacc, w2)
    return (out, ei, edge_attr)
